# baseline (speedup 1.0000x reference)
import functools

import jax
import jax.numpy as jnp
from jax import lax
from jax.experimental import pallas as pl
from jax.experimental.pallas import tpu as pltpu
from jax.experimental.pallas import tpu_sc as plsc

N = 10000
D = 128
E = 320000
NC, NS = 2, 16
NW = NC * NS
K = 128
NCHUNK = 80
EPW = NCHUNK * K
EPAD = EPW * NW
NJUNK = 16
NACC = 10112
ZROWS = NACC // NS

_MESH = plsc.VectorSubcoreMesh(
    core_axis_name="c", subcore_axis_name="s", num_cores=NC, num_subcores=NS
)

_PART_SDS = jax.ShapeDtypeStruct((NC, NACC, D), jnp.float32)


def _acc_prologue(zeros_hbm, acc, tid):
    pltpu.sync_copy(zeros_hbm, acc.at[pl.ds(tid * ZROWS, ZROWS)])
    plsc.subcore_barrier()


def _acc_epilogue(out_hbm, acc, cid, tid):
    plsc.subcore_barrier()
    base = tid * ZROWS
    pltpu.sync_copy(acc.at[pl.ds(base, ZROWS)],
                    out_hbm.at[cid, pl.ds(base, ZROWS)])


@functools.partial(
    pl.kernel,
    out_type=_PART_SDS,
    mesh=_MESH,
    scratch_types=[
        pltpu.VMEM((NCHUNK, K), jnp.int32),
        pltpu.VMEM((K, D), jnp.float32),
        pltpu.VMEM_SHARED((NACC, D), jnp.float32),
    ],
)
def _deg_kernel(dstp_hbm, ones_hbm, zeros_hbm, out_hbm, dst_v, ones_v, acc):
    cid = lax.axis_index("c")
    tid = lax.axis_index("s")
    wid = tid * NC + cid

    _acc_prologue(zeros_hbm, acc, tid)

    pltpu.sync_copy(dstp_hbm.at[wid], dst_v)
    pltpu.sync_copy(ones_hbm, ones_v)

    def chunk(j, c):
        pltpu.sync_copy(ones_v, acc.at[dst_v.at[j]], add=True)
        return c

    lax.fori_loop(0, NCHUNK, chunk, 0)

    _acc_epilogue(out_hbm, acc, cid, tid)


KS = 64
NCH = EPW // KS
PH = NCH // 4
NBUF = 4


@functools.partial(
    pl.kernel,
    out_type=_PART_SDS,
    mesh=_MESH,
    scratch_types=[
        pltpu.VMEM((PH, KS), jnp.int32),
        pltpu.VMEM((PH, KS), jnp.int32),
        pltpu.VMEM((KS, D), jnp.float32),
        pltpu.VMEM((KS, D), jnp.float32),
        pltpu.VMEM((KS, D), jnp.float32),
        pltpu.VMEM((KS, D), jnp.float32),
        pltpu.VMEM_SHARED((NACC, D), jnp.float32),
        pltpu.SemaphoreType.DMA,
        pltpu.SemaphoreType.DMA,
        pltpu.SemaphoreType.DMA,
        pltpu.SemaphoreType.DMA,
    ],
)
def _scat_kernel(y_hbm, srcp_hbm, dstp_hbm, zeros_hbm, out_hbm, src_v, dst_v,
                 r0, r1, r2, r3, acc, s0, s1, s2, s3):
    cid = lax.axis_index("c")
    tid = lax.axis_index("s")
    wid = tid * NC + cid

    _acc_prologue(zeros_hbm, acc, tid)

    bufs = (r0, r1, r2, r3)
    sems = (s0, s1, s2, s3)

    def gat(c, b):
        return pltpu.make_async_copy(y_hbm.at[src_v.at[c]], bufs[b], sems[b])

    def scat(c, b):
        gat(c, b).wait()
        pltpu.sync_copy(bufs[b], acc.at[dst_v.at[c]], add=True)

    for p in range(4):
        pltpu.sync_copy(srcp_hbm.at[wid, pl.ds(p * PH, PH)], src_v)
        pltpu.sync_copy(dstp_hbm.at[wid, pl.ds(p * PH, PH)], dst_v)

        for b in range(NBUF - 1):
            gat(b, b).start()

        def body(j, c):
            c0 = NBUF * j
            for b in range(NBUF):
                scat(c0 + b, b)
                gat(c0 + b + NBUF - 1, (b + NBUF - 1) % NBUF).start()
            return c

        lax.fori_loop(0, PH // NBUF - 1, body, 0)

        scat(PH - 4, 0)
        gat(PH - 1, (PH - 1) % NBUF).start()
        scat(PH - 3, 1)
        scat(PH - 2, 2)
        scat(PH - 1, 3)

    _acc_epilogue(out_hbm, acc, cid, tid)


_GRID = N // 1000
_ROWS = pl.BlockSpec((1000, D), lambda i: (i, 0))
_SPB = pl.BlockSpec((NC, 1000, D), lambda i: (0, i, 0))
_WMAT = pl.BlockSpec((D, D), lambda i: (0, 0))
_BIAS = pl.BlockSpec((8, D), lambda i: (0, 0))
_OUT_SDS = jax.ShapeDtypeStruct((N, D), jnp.float32)


def _dinv(dp_ref):
    return lax.rsqrt(1.0 + dp_ref[0] + dp_ref[1])


def _tc_first_body(dp_ref, x_ref, w_ref, y_ref):
    dv = _dinv(dp_ref)
    y_ref[...] = jnp.dot(x_ref[...], w_ref[...],
                         preferred_element_type=jnp.float32) * dv


_tc_first = pl.pallas_call(
    _tc_first_body,
    grid=(_GRID,),
    in_specs=[_SPB, _ROWS, _WMAT],
    out_specs=_ROWS,
    out_shape=_OUT_SDS,
)


def _tc_mid_body(dp_ref, sp_ref, y_ref, b_ref, w_ref, o_ref):
    dv = _dinv(dp_ref)
    h = dv * (sp_ref[0] + sp_ref[1] + y_ref[...]) + b_ref[0:1, :]
    h = jnp.maximum(h, 0.0)
    o_ref[...] = jnp.dot(h, w_ref[...], preferred_element_type=jnp.float32) * dv


_tc_mid = pl.pallas_call(
    _tc_mid_body,
    grid=(_GRID,),
    in_specs=[_SPB, _SPB, _ROWS, _BIAS, _WMAT],
    out_specs=_ROWS,
    out_shape=_OUT_SDS,
)


def _tc_last_body(dp_ref, sp_ref, y_ref, b3_ref, wd1_ref, bd1_ref, wd2_ref,
                  bd2_ref, o_ref):
    dv = _dinv(dp_ref)
    h = dv * (sp_ref[0] + sp_ref[1] + y_ref[...]) + b3_ref[0:1, :]
    h = jnp.maximum(h, 0.0)
    h = jnp.dot(h, wd1_ref[...], preferred_element_type=jnp.float32)
    h = jnp.maximum(h + bd1_ref[0:1, :], 0.0)
    z = jnp.dot(h, wd2_ref[...], preferred_element_type=jnp.float32)
    o_ref[...] = jax.nn.sigmoid(z + bd2_ref[0:1, :])


_tc_last = pl.pallas_call(
    _tc_last_body,
    grid=(_GRID,),
    in_specs=[_SPB, _SPB, _ROWS, _BIAS, _WMAT, _BIAS, _WMAT, _BIAS],
    out_specs=_ROWS,
    out_shape=_OUT_SDS,
)


def _bias8(b):
    return jnp.broadcast_to(b.reshape(1, -1), (8, b.shape[0]))


def kernel(x, edge_index, W1, b1, W2, b2, W3, b3, Wd1, bd1, Wd2, bd2):
    src = edge_index[0].astype(jnp.int32)
    dst = edge_index[1].astype(jnp.int32)
    npad = EPAD - E
    pidx = jnp.arange(npad, dtype=jnp.int32)
    srcf = jnp.concatenate([src, (pidx * 97) % N])
    dstf = jnp.concatenate([dst, N + (pidx % NJUNK)])
    srcp = srcf.reshape(NW, NCH, KS)
    dstp = dstf.reshape(NW, NCH, KS)
    dstp_deg = dstf.reshape(NW, NCHUNK, K)
    ones_t = jnp.ones((K, D), jnp.float32)
    zeros_t = jnp.zeros((ZROWS, D), jnp.float32)

    dp = _deg_kernel(dstp_deg, ones_t, zeros_t)

    b1b, b2b, b3b, bd1b = _bias8(b1), _bias8(b2), _bias8(b3), _bias8(bd1)
    wd2p = jnp.pad(Wd2, ((0, 0), (0, D - Wd2.shape[1])))
    bd2b = jnp.broadcast_to(bd2.reshape(1, 1), (8, D))

    y1 = _tc_first(dp, x, W1)
    s1 = _scat_kernel(y1, srcp, dstp, zeros_t)
    y2 = _tc_mid(dp, s1, y1, b1b, W2)
    s2 = _scat_kernel(y2, srcp, dstp, zeros_t)
    y3 = _tc_mid(dp, s2, y2, b2b, W3)
    s3 = _scat_kernel(y3, srcp, dstp, zeros_t)
    out = _tc_last(dp, s3, y3, b3b, Wd1, bd1b, wd2p, bd2b)
    return out[:, :1]


# speedup vs baseline: 1.0125x; 1.0125x over previous
import functools

import jax
import jax.numpy as jnp
from jax import lax
from jax.experimental import pallas as pl
from jax.experimental.pallas import tpu as pltpu
from jax.experimental.pallas import tpu_sc as plsc

N = 10000
D = 128
DW = 128
E = 320000
NC, NS = 2, 16
NW = NC * NS
K = 128
NCHUNK = 80
EPW = NCHUNK * K
EPAD = EPW * NW
NJUNK = 16
NACC = 10112
ZROWS = NACC // NS

_MESH = plsc.VectorSubcoreMesh(
    core_axis_name="c", subcore_axis_name="s", num_cores=NC, num_subcores=NS
)

_PART_SDS = jax.ShapeDtypeStruct((NC, NACC, D), jnp.float32)
_DEG_SDS = jax.ShapeDtypeStruct((NC, NACC, DW), jnp.float32)


def _acc_prologue(zeros_hbm, acc, stage_v, tid):
    pltpu.sync_copy(zeros_hbm, stage_v)
    for m in range(4):
        pltpu.sync_copy(stage_v, acc.at[pl.ds(tid * ZROWS + m * K, K)])
    tail = ZROWS - 4 * K
    pltpu.sync_copy(stage_v.at[pl.ds(0, tail)],
                    acc.at[pl.ds(tid * ZROWS + 4 * K, tail)])
    plsc.subcore_barrier()


def _acc_epilogue(out_hbm, acc, stage_v, cid, tid):
    plsc.subcore_barrier()
    base = tid * ZROWS
    for m in range(4):
        pltpu.sync_copy(acc.at[pl.ds(base + m * K, K)], stage_v)
        pltpu.sync_copy(stage_v, out_hbm.at[cid, pl.ds(base + m * K, K)])
    tail = ZROWS - 4 * K
    pltpu.sync_copy(acc.at[pl.ds(base + 4 * K, tail)],
                    stage_v.at[pl.ds(0, tail)])
    pltpu.sync_copy(stage_v.at[pl.ds(0, tail)],
                    out_hbm.at[cid, pl.ds(base + 4 * K, tail)])


@functools.partial(
    pl.kernel,
    out_type=_DEG_SDS,
    mesh=_MESH,
    scratch_types=[
        pltpu.VMEM((NCHUNK, K), jnp.int32),
        pltpu.VMEM((K, DW), jnp.float32),
        pltpu.VMEM_SHARED((NACC, DW), jnp.float32),
    ],
)
def _deg_kernel(dstp_hbm, ones_hbm, zeros_hbm, out_hbm, dst_v, ones_v, acc):
    cid = lax.axis_index("c")
    tid = lax.axis_index("s")
    wid = tid * NC + cid

    _acc_prologue(zeros_hbm, acc, ones_v, tid)

    pltpu.sync_copy(dstp_hbm.at[wid], dst_v)
    pltpu.sync_copy(ones_hbm, ones_v)

    def chunk(j, c):
        pltpu.sync_copy(ones_v, acc.at[dst_v.at[j]], add=True)
        return c

    lax.fori_loop(0, NCHUNK, chunk, 0)

    _acc_epilogue(out_hbm, acc, ones_v, cid, tid)


HC2 = NCHUNK // 2


@functools.partial(
    pl.kernel,
    out_type=_PART_SDS,
    mesh=_MESH,
    scratch_types=[
        pltpu.VMEM((HC2, K), jnp.int32),
        pltpu.VMEM((HC2, K), jnp.int32),
        pltpu.VMEM((K, D), jnp.float32),
        pltpu.VMEM((K, D), jnp.float32),
        pltpu.VMEM_SHARED((NACC, D), jnp.float32),
        pltpu.SemaphoreType.DMA,
        pltpu.SemaphoreType.DMA,
    ],
)
def _scat_kernel(y_hbm, srcp_hbm, dstp_hbm, zeros_hbm, out_hbm, src_v, dst_v,
                 rows_a, rows_b, acc, sem_a, sem_b):
    cid = lax.axis_index("c")
    tid = lax.axis_index("s")
    wid = tid * NC + cid

    _acc_prologue(zeros_hbm, acc, rows_a, tid)

    def gat_a(c):
        return pltpu.make_async_copy(y_hbm.at[src_v.at[c]], rows_a, sem_a)

    def gat_b(c):
        return pltpu.make_async_copy(y_hbm.at[src_v.at[c]], rows_b, sem_b)

    for p in range(2):
        pltpu.sync_copy(srcp_hbm.at[wid, pl.ds(p * HC2, HC2)], src_v)
        pltpu.sync_copy(dstp_hbm.at[wid, pl.ds(p * HC2, HC2)], dst_v)

        gat_a(0).start()

        def body(j2, c):
            c0 = 2 * j2
            gat_b(c0 + 1).start()
            gat_a(c0).wait()
            pltpu.sync_copy(rows_a, acc.at[dst_v.at[c0]], add=True)
            gat_a(c0 + 2).start()
            gat_b(c0 + 1).wait()
            pltpu.sync_copy(rows_b, acc.at[dst_v.at[c0 + 1]], add=True)
            return c

        lax.fori_loop(0, HC2 // 2 - 1, body, 0)

        gat_b(HC2 - 1).start()
        gat_a(HC2 - 2).wait()
        pltpu.sync_copy(rows_a, acc.at[dst_v.at[HC2 - 2]], add=True)
        gat_b(HC2 - 1).wait()
        pltpu.sync_copy(rows_b, acc.at[dst_v.at[HC2 - 1]], add=True)

    _acc_epilogue(out_hbm, acc, rows_a, cid, tid)


_GRID = N // 1000
_ROWS = pl.BlockSpec((1000, D), lambda i: (i, 0))
_SPB = pl.BlockSpec((NC, 1000, D), lambda i: (0, i, 0))
_DPB = pl.BlockSpec((NC, 1000, DW), lambda i: (0, i, 0))
_WMAT = pl.BlockSpec((D, D), lambda i: (0, 0))
_BIAS = pl.BlockSpec((8, D), lambda i: (0, 0))
_OUT_SDS = jax.ShapeDtypeStruct((N, D), jnp.float32)


def _dinv(dp_ref):
    return lax.rsqrt(1.0 + dp_ref[0][:, 0:1] + dp_ref[1][:, 0:1])


def _tc_first_body(dp_ref, x_ref, w_ref, y_ref):
    dv = _dinv(dp_ref)
    y_ref[...] = jnp.dot(x_ref[...], w_ref[...],
                         preferred_element_type=jnp.float32) * dv


_tc_first = pl.pallas_call(
    _tc_first_body,
    grid=(_GRID,),
    in_specs=[_DPB, _ROWS, _WMAT],
    out_specs=_ROWS,
    out_shape=_OUT_SDS,
)


def _tc_mid_body(dp_ref, sp_ref, y_ref, b_ref, w_ref, o_ref):
    dv = _dinv(dp_ref)
    h = dv * (sp_ref[0] + sp_ref[1] + y_ref[...]) + b_ref[0:1, :]
    h = jnp.maximum(h, 0.0)
    o_ref[...] = jnp.dot(h, w_ref[...], preferred_element_type=jnp.float32) * dv


_tc_mid = pl.pallas_call(
    _tc_mid_body,
    grid=(_GRID,),
    in_specs=[_DPB, _SPB, _ROWS, _BIAS, _WMAT],
    out_specs=_ROWS,
    out_shape=_OUT_SDS,
)


def _tc_last_body(dp_ref, sp_ref, y_ref, b3_ref, wd1_ref, bd1_ref, wd2_ref,
                  bd2_ref, o_ref):
    dv = _dinv(dp_ref)
    h = dv * (sp_ref[0] + sp_ref[1] + y_ref[...]) + b3_ref[0:1, :]
    h = jnp.maximum(h, 0.0)
    h = jnp.dot(h, wd1_ref[...], preferred_element_type=jnp.float32)
    h = jnp.maximum(h + bd1_ref[0:1, :], 0.0)
    z = jnp.dot(h, wd2_ref[...], preferred_element_type=jnp.float32)
    o_ref[...] = jax.nn.sigmoid(z + bd2_ref[0:1, :])


_tc_last = pl.pallas_call(
    _tc_last_body,
    grid=(_GRID,),
    in_specs=[_DPB, _SPB, _ROWS, _BIAS, _WMAT, _BIAS, _WMAT, _BIAS],
    out_specs=_ROWS,
    out_shape=_OUT_SDS,
)


def _bias8(b):
    return jnp.broadcast_to(b.reshape(1, -1), (8, b.shape[0]))


def kernel(x, edge_index, W1, b1, W2, b2, W3, b3, Wd1, bd1, Wd2, bd2):
    src = edge_index[0].astype(jnp.int32)
    dst = edge_index[1].astype(jnp.int32)
    npad = EPAD - E
    pidx = jnp.arange(npad, dtype=jnp.int32)
    srcp = jnp.concatenate([src, (pidx * 97) % N]).reshape(NW, NCHUNK, K)
    dstp = jnp.concatenate([dst, N + (pidx % NJUNK)]).reshape(NW, NCHUNK, K)
    ones8_t = jnp.ones((K, DW), jnp.float32)
    zeros8_t = jnp.zeros((K, DW), jnp.float32)
    zeros_t = jnp.zeros((K, D), jnp.float32)

    dp = _deg_kernel(dstp, ones8_t, zeros8_t)

    b1b, b2b, b3b, bd1b = _bias8(b1), _bias8(b2), _bias8(b3), _bias8(bd1)
    wd2p = jnp.pad(Wd2, ((0, 0), (0, D - Wd2.shape[1])))
    bd2b = jnp.broadcast_to(bd2.reshape(1, 1), (8, D))

    y1 = _tc_first(dp, x, W1)
    s1 = _scat_kernel(y1, srcp, dstp, zeros_t)
    y2 = _tc_mid(dp, s1, y1, b1b, W2)
    s2 = _scat_kernel(y2, srcp, dstp, zeros_t)
    y3 = _tc_mid(dp, s2, y2, b2b, W3)
    s3 = _scat_kernel(y3, srcp, dstp, zeros_t)
    out = _tc_last(dp, s3, y3, b3b, Wd1, bd1b, wd2p, bd2b)
    return out[:, :1]
